# Initial kernel scaffold; baseline (speedup 1.0000x reference)
import jax
import jax.numpy as jnp
from jax.experimental import pallas as pl


def kernel(x, W1, W2):
    raise NotImplementedError("write your pallas kernel here")


# speedup vs baseline: 20.1180x; 20.1180x over previous
import jax
import jax.numpy as jnp
from jax.experimental import pallas as pl
from jax.experimental.pallas import tpu as pltpu

_INPUT_DIM = 3072
_HIDDEN_DIM = 4096
_TIME_STEPS = 30
_TAU_MEM = 100.0
_THRESHOLD = 0.5
_INPUT_SCALE = 16.0


def _lif_body(x_ref, w1_ref, w2_ref, d_ref,
              mr_ref, m1_ref, m2_ref,
              vr_ref, v1_ref, v2_ref):
    decay = d_ref[0, 0]
    x = x_ref[...]
    vr_ref[...] = jnp.zeros_like(vr_ref)
    v1_ref[...] = jnp.zeros_like(v1_ref)
    v2_ref[...] = jnp.zeros_like(v2_ref)
    mr_ref[...] = jnp.zeros_like(mr_ref)
    m1_ref[...] = jnp.zeros_like(m1_ref)
    m2_ref[...] = jnp.zeros_like(m2_ref)

    def step(_, carry):
        vr = vr_ref[...] * decay + x
        sr = (vr > _THRESHOLD).astype(jnp.float32)
        vr_ref[...] = vr * (1.0 - sr)
        a1 = jnp.dot(sr.astype(w1_ref.dtype), w1_ref[...],
                     preferred_element_type=jnp.float32)
        v1 = v1_ref[...] * decay + a1
        s1 = (v1 > _THRESHOLD).astype(jnp.float32)
        v1_ref[...] = v1 * (1.0 - s1)
        a2 = jnp.dot(s1.astype(w2_ref.dtype), w2_ref[...],
                     preferred_element_type=jnp.float32)
        v2 = v2_ref[...] * decay + a2
        s2 = (v2 > _THRESHOLD).astype(jnp.float32)
        v2_ref[...] = v2 * (1.0 - s2)
        mr_ref[...] += sr
        m1_ref[...] += s1
        m2_ref[...] += s2
        return carry

    jax.lax.fori_loop(0, _TIME_STEPS, step, 0)

    denom = jnp.float32(_TIME_STEPS)
    mr_ref[...] = mr_ref[...] / denom
    m1_ref[...] = m1_ref[...] / denom
    m2_ref[...] = m2_ref[...] / denom


def kernel(x, W1, W2):
    B = x.shape[0]
    xmax = jnp.max(x)
    xn = jnp.where(xmax > 0, x / xmax, x) * _INPUT_SCALE
    decay = jnp.exp(jnp.float32(-1.0 / _TAU_MEM)).reshape(1, 1)
    w1 = W1.astype(jnp.bfloat16)
    w2 = W2.astype(jnp.bfloat16)
    f32 = jnp.float32
    out_shape = (
        jax.ShapeDtypeStruct((B, _INPUT_DIM), f32),
        jax.ShapeDtypeStruct((B, _HIDDEN_DIM), f32),
        jax.ShapeDtypeStruct((B, _HIDDEN_DIM), f32),
    )
    mr, m1, m2 = pl.pallas_call(
        _lif_body,
        out_shape=out_shape,
        scratch_shapes=[
            pltpu.VMEM((B, _INPUT_DIM), f32),
            pltpu.VMEM((B, _HIDDEN_DIM), f32),
            pltpu.VMEM((B, _HIDDEN_DIM), f32),
        ],
        compiler_params=pltpu.CompilerParams(
            vmem_limit_bytes=64 * 1024 * 1024,
        ),
    )(xn, w1, w2, decay)
    return (mr, m1, m2)


# speedup vs baseline: 36.4416x; 1.8114x over previous
import jax
import jax.numpy as jnp
from jax.experimental import pallas as pl
from jax.experimental.pallas import tpu as pltpu

_INPUT_DIM = 3072
_HIDDEN_DIM = 4096
_TIME_STEPS = 30
_TAU_MEM = 100.0
_THRESHOLD = 0.5
_INPUT_SCALE = 16.0
_NBLK = 8
_NB = _HIDDEN_DIM // _NBLK


def _body_a(x_ref, w1_ref, d_ref, mr_ref, m1_ref, s1a_ref,
            sra_ref, a_ref, vr_ref, v1_ref):
    j = pl.program_id(0)
    decay = d_ref[0, 0]
    T = _TIME_STEPS
    B = x_ref.shape[0]
    K = x_ref.shape[1]

    @pl.when(j == 0)
    def _retina():
        x = x_ref[...]
        vr_ref[...] = jnp.zeros_like(vr_ref)
        mr_ref[...] = jnp.zeros_like(mr_ref)

        def rstep(t, c):
            vr = vr_ref[...] * decay + x
            sr = (vr > _THRESHOLD).astype(jnp.float32)
            vr_ref[...] = vr * (1.0 - sr)
            mr_ref[...] += sr
            sra_ref[t] = sr.astype(jnp.bfloat16)
            return c

        jax.lax.fori_loop(0, T, rstep, 0)
        mr_ref[...] = mr_ref[...] / jnp.float32(T)

    lhs = sra_ref[...].reshape(T * B, K)
    a_ref[...] = jnp.dot(lhs, w1_ref[...],
                         preferred_element_type=jnp.float32).reshape(T, B, _NB)
    v1_ref[...] = jnp.zeros_like(v1_ref)
    m1_ref[...] = jnp.zeros_like(m1_ref)

    def lstep(t, c):
        v1 = v1_ref[...] * decay + a_ref[t]
        s1 = (v1 > _THRESHOLD).astype(jnp.float32)
        v1_ref[...] = v1 * (1.0 - s1)
        m1_ref[...] += s1
        s1a_ref[t] = s1.astype(jnp.bfloat16)
        return c

    jax.lax.fori_loop(0, T, lstep, 0)
    m1_ref[...] = m1_ref[...] / jnp.float32(T)


def _body_b(s1a_ref, w2_ref, d_ref, m2_ref, a_ref, v2_ref):
    decay = d_ref[0, 0]
    T = s1a_ref.shape[0]
    B = s1a_ref.shape[1]
    K = s1a_ref.shape[2]

    lhs = s1a_ref[...].reshape(T * B, K)
    a_ref[...] = jnp.dot(lhs, w2_ref[...],
                         preferred_element_type=jnp.float32).reshape(T, B, _NB)
    v2_ref[...] = jnp.zeros_like(v2_ref)
    m2_ref[...] = jnp.zeros_like(m2_ref)

    def lstep(t, c):
        v2 = v2_ref[...] * decay + a_ref[t]
        s2 = (v2 > _THRESHOLD).astype(jnp.float32)
        v2_ref[...] = v2 * (1.0 - s2)
        m2_ref[...] += s2
        return c

    jax.lax.fori_loop(0, T, lstep, 0)
    m2_ref[...] = m2_ref[...] / jnp.float32(T)


def kernel(x, W1, W2):
    B = x.shape[0]
    T = _TIME_STEPS
    f32 = jnp.float32
    bf16 = jnp.bfloat16
    xmax = jnp.max(x)
    xn = jnp.where(xmax > 0, x / xmax, x) * _INPUT_SCALE
    decay = jnp.exp(jnp.float32(-1.0 / _TAU_MEM)).reshape(1, 1)
    w1 = W1.astype(bf16)
    w2 = W2.astype(bf16)

    mr, m1, s1a = pl.pallas_call(
        _body_a,
        grid=(_NBLK,),
        in_specs=[
            pl.BlockSpec((B, _INPUT_DIM), lambda j: (0, 0)),
            pl.BlockSpec((_INPUT_DIM, _NB), lambda j: (0, j)),
            pl.BlockSpec((1, 1), lambda j: (0, 0)),
        ],
        out_specs=[
            pl.BlockSpec((B, _INPUT_DIM), lambda j: (0, 0)),
            pl.BlockSpec((B, _NB), lambda j: (0, j)),
            pl.BlockSpec((T, B, _NB), lambda j: (0, 0, j)),
        ],
        out_shape=[
            jax.ShapeDtypeStruct((B, _INPUT_DIM), f32),
            jax.ShapeDtypeStruct((B, _HIDDEN_DIM), f32),
            jax.ShapeDtypeStruct((T, B, _HIDDEN_DIM), bf16),
        ],
        scratch_shapes=[
            pltpu.VMEM((T, B, _INPUT_DIM), bf16),
            pltpu.VMEM((T, B, _NB), f32),
            pltpu.VMEM((B, _INPUT_DIM), f32),
            pltpu.VMEM((B, _NB), f32),
        ],
        compiler_params=pltpu.CompilerParams(
            dimension_semantics=("arbitrary",),
            vmem_limit_bytes=64 * 1024 * 1024,
        ),
    )(xn, w1, decay)

    (m2,) = pl.pallas_call(
        _body_b,
        grid=(_NBLK,),
        in_specs=[
            pl.BlockSpec((T, B, _HIDDEN_DIM), lambda j: (0, 0, 0)),
            pl.BlockSpec((_HIDDEN_DIM, _NB), lambda j: (0, j)),
            pl.BlockSpec((1, 1), lambda j: (0, 0)),
        ],
        out_specs=[
            pl.BlockSpec((B, _NB), lambda j: (0, j)),
        ],
        out_shape=[
            jax.ShapeDtypeStruct((B, _HIDDEN_DIM), f32),
        ],
        scratch_shapes=[
            pltpu.VMEM((T, B, _NB), f32),
            pltpu.VMEM((B, _NB), f32),
        ],
        compiler_params=pltpu.CompilerParams(
            dimension_semantics=("arbitrary",),
            vmem_limit_bytes=64 * 1024 * 1024,
        ),
    )(s1a, w2, decay)

    return (mr, m1, m2)


# speedup vs baseline: 48.3497x; 1.3268x over previous
import jax
import jax.numpy as jnp
from jax.experimental import pallas as pl
from jax.experimental.pallas import tpu as pltpu

_INPUT_DIM = 3072
_HIDDEN_DIM = 4096
_TIME_STEPS = 30
_TAU_MEM = 100.0
_THRESHOLD = 0.5
_INPUT_SCALE = 16.0
_NBLK = 8
_NB = _HIDDEN_DIM // _NBLK


def _body_a(x_ref, w1_ref, d_ref, mr_ref, m1_ref, s1a_ref,
            sra_ref, a_ref, vr_ref, v1_ref):
    j = pl.program_id(0)
    decay = d_ref[0, 0]
    T = _TIME_STEPS
    B = x_ref.shape[0]
    K = x_ref.shape[1]

    @pl.when(j == 0)
    def _retina():
        x = x_ref[...]
        vr_ref[...] = jnp.zeros_like(vr_ref)
        mr_ref[...] = jnp.zeros_like(mr_ref)

        def rstep(t, c):
            vr = vr_ref[...] * decay + x
            sr = (vr > _THRESHOLD).astype(jnp.float32)
            vr_ref[...] = vr * (1.0 - sr)
            mr_ref[...] += sr
            sra_ref[t] = sr.astype(jnp.bfloat16)
            return c

        jax.lax.fori_loop(0, T, rstep, 0)
        mr_ref[...] = mr_ref[...] / jnp.float32(T)

    @pl.when(j < _NBLK)
    def _dot():
        lhs = sra_ref[...].reshape(T * B, K)
        w1b = w1_ref[...].astype(jnp.bfloat16)
        a_ref[j % 2] = jnp.dot(
            lhs, w1b, preferred_element_type=jnp.float32).reshape(T, B, _NB)

    @pl.when(j > 0)
    def _recur():
        v1_ref[...] = jnp.zeros_like(v1_ref)
        m1_ref[...] = jnp.zeros_like(m1_ref)
        buf = (j - 1) % 2

        def lstep(t, c):
            v1 = v1_ref[...] * decay + a_ref[buf, t]
            s1 = (v1 > _THRESHOLD).astype(jnp.float32)
            v1_ref[...] = v1 * (1.0 - s1)
            m1_ref[...] += s1
            s1a_ref[t] = s1.astype(jnp.bfloat16)
            return c

        jax.lax.fori_loop(0, T, lstep, 0)
        m1_ref[...] = m1_ref[...] / jnp.float32(T)


def _body_b(s1a_ref, w2_ref, d_ref, m2_ref, a_ref, v2_ref):
    j = pl.program_id(0)
    decay = d_ref[0, 0]
    T = s1a_ref.shape[0]
    B = s1a_ref.shape[1]
    K = s1a_ref.shape[2]

    @pl.when(j < _NBLK)
    def _dot():
        lhs = s1a_ref[...].reshape(T * B, K)
        w2b = w2_ref[...].astype(jnp.bfloat16)
        a_ref[j % 2] = jnp.dot(
            lhs, w2b, preferred_element_type=jnp.float32).reshape(T, B, _NB)

    @pl.when(j > 0)
    def _recur():
        v2_ref[...] = jnp.zeros_like(v2_ref)
        m2_ref[...] = jnp.zeros_like(m2_ref)
        buf = (j - 1) % 2

        def lstep(t, c):
            v2 = v2_ref[...] * decay + a_ref[buf, t]
            s2 = (v2 > _THRESHOLD).astype(jnp.float32)
            v2_ref[...] = v2 * (1.0 - s2)
            m2_ref[...] += s2
            return c

        jax.lax.fori_loop(0, T, lstep, 0)
        m2_ref[...] = m2_ref[...] / jnp.float32(T)


def kernel(x, W1, W2):
    B = x.shape[0]
    T = _TIME_STEPS
    f32 = jnp.float32
    bf16 = jnp.bfloat16
    xmax = jnp.max(x)
    xn = jnp.where(xmax > 0, x / xmax, x) * _INPUT_SCALE
    decay = jnp.exp(jnp.float32(-1.0 / _TAU_MEM)).reshape(1, 1)

    prev = lambda j: jnp.maximum(j, 1) - 1
    last = lambda j: jnp.minimum(j, _NBLK - 1)

    mr, m1, s1a = pl.pallas_call(
        _body_a,
        grid=(_NBLK + 1,),
        in_specs=[
            pl.BlockSpec((B, _INPUT_DIM), lambda j: (0, 0)),
            pl.BlockSpec((_INPUT_DIM, _NB), lambda j: (0, last(j))),
            pl.BlockSpec((1, 1), lambda j: (0, 0)),
        ],
        out_specs=[
            pl.BlockSpec((B, _INPUT_DIM), lambda j: (0, 0)),
            pl.BlockSpec((B, _NB), lambda j: (0, prev(j))),
            pl.BlockSpec((T, B, _NB), lambda j: (0, 0, prev(j))),
        ],
        out_shape=[
            jax.ShapeDtypeStruct((B, _INPUT_DIM), f32),
            jax.ShapeDtypeStruct((B, _HIDDEN_DIM), f32),
            jax.ShapeDtypeStruct((T, B, _HIDDEN_DIM), bf16),
        ],
        scratch_shapes=[
            pltpu.VMEM((T, B, _INPUT_DIM), bf16),
            pltpu.VMEM((2, T, B, _NB), f32),
            pltpu.VMEM((B, _INPUT_DIM), f32),
            pltpu.VMEM((B, _NB), f32),
        ],
        compiler_params=pltpu.CompilerParams(
            dimension_semantics=("arbitrary",),
            vmem_limit_bytes=64 * 1024 * 1024,
        ),
    )(xn, W1, decay)

    (m2,) = pl.pallas_call(
        _body_b,
        grid=(_NBLK + 1,),
        in_specs=[
            pl.BlockSpec((T, B, _HIDDEN_DIM), lambda j: (0, 0, 0)),
            pl.BlockSpec((_HIDDEN_DIM, _NB), lambda j: (0, last(j))),
            pl.BlockSpec((1, 1), lambda j: (0, 0)),
        ],
        out_specs=[
            pl.BlockSpec((B, _NB), lambda j: (0, prev(j))),
        ],
        out_shape=[
            jax.ShapeDtypeStruct((B, _HIDDEN_DIM), f32),
        ],
        scratch_shapes=[
            pltpu.VMEM((2, T, B, _NB), f32),
            pltpu.VMEM((B, _NB), f32),
        ],
        compiler_params=pltpu.CompilerParams(
            dimension_semantics=("arbitrary",),
            vmem_limit_bytes=64 * 1024 * 1024,
        ),
    )(s1a, W2, decay)

    return (mr, m1, m2)
